# baseline (speedup 1.0000x reference)
import functools

import jax
import jax.numpy as jnp
from jax import lax
from jax.experimental import pallas as pl
from jax.experimental.pallas import tpu as pltpu
from jax.experimental.pallas import tpu_sc as plsc

N = 10000
E = 320000
D = 128

NUM_CORES = 2
NUM_SUBCORES = 16
NW = NUM_CORES * NUM_SUBCORES
BLK = 64
NB = 160
G = 8
NG = NB // G
E_PAD = NW * NB * BLK
PAD_ROWS = 128
N_PAD = 10240
ROWS_PER_TILE = N_PAD // NUM_SUBCORES

_mesh = plsc.VectorSubcoreMesh(core_axis_name="c", subcore_axis_name="s")


NR = N_PAD // 128


@functools.partial(
    pl.kernel,
    out_type=jax.ShapeDtypeStruct((NUM_CORES, NR, 128), jnp.float32),
    mesh=_mesh,
    compiler_params=pltpu.CompilerParams(needs_layout_passes=False),
    scratch_types=[
        pltpu.VMEM((NB, BLK), jnp.int32),
        pltpu.VMEM((NR, 128), jnp.float32),
        pltpu.VMEM((1, NR), jnp.int32),
        pltpu.VMEM_SHARED((NR, 128), jnp.float32),
    ],
)
def _sc_degree(dst2d, zeros_hbm, iota_hbm, deg_out, dst_v, hist_v, iota_v,
               deg_sp):
    c = lax.axis_index("c")
    s = lax.axis_index("s")
    wid = c * NUM_SUBCORES + s
    pltpu.sync_copy(dst2d.at[wid], dst_v)
    pltpu.sync_copy(zeros_hbm.at[pl.ds(0, NR)], hist_v)
    pltpu.sync_copy(iota_hbm, iota_v)

    @pl.when(s == 0)
    def _():
        pltpu.sync_copy(zeros_hbm.at[pl.ds(0, NR)], deg_sp)

    def row(r, carry):
        for k in range(BLK // 16):
            idx = dst_v[r, pl.ds(k * 16, 16)]
            cnt, last = plsc.scan_count(idx)
            plsc.addupdate_scatter(
                hist_v,
                [lax.shift_right_logical(idx, 7), lax.bitwise_and(idx, 127)],
                cnt.astype(jnp.float32), mask=last)
        return carry

    lax.fori_loop(0, NB, row, 0)
    plsc.subcore_barrier()
    pltpu.sync_copy(hist_v, deg_sp.at[iota_v.at[0]], add=True)
    plsc.subcore_barrier()

    @pl.when(s == 0)
    def _():
        pltpu.sync_copy(deg_sp, deg_out.at[c])


@functools.partial(
    pl.kernel,
    out_type=jax.ShapeDtypeStruct((NUM_CORES, N_PAD, D), jnp.float32),
    mesh=_mesh,
    scratch_types=[
        pltpu.VMEM((G, BLK), jnp.int32),
        pltpu.VMEM((G, BLK), jnp.int32),
        pltpu.VMEM((G, BLK), jnp.int32),
        pltpu.VMEM((G, BLK), jnp.int32),
        pltpu.VMEM((BLK, D), jnp.float32),
        pltpu.VMEM((BLK, D), jnp.float32),
        pltpu.VMEM((BLK, D), jnp.float32),
        pltpu.VMEM((BLK, D), jnp.float32),
        pltpu.VMEM((BLK, D), jnp.float32),
        pltpu.VMEM_SHARED((N_PAD, D), jnp.float32),
        pltpu.SemaphoreType.DMA,
        pltpu.SemaphoreType.DMA,
        pltpu.SemaphoreType.DMA,
        pltpu.SemaphoreType.DMA,
        pltpu.SemaphoreType.DMA,
        pltpu.SemaphoreType.DMA,
        pltpu.SemaphoreType.DMA,
        pltpu.SemaphoreType.DMA,
        pltpu.SemaphoreType.DMA,
        pltpu.SemaphoreType.DMA,
        pltpu.SemaphoreType.DMA,
        pltpu.SemaphoreType.DMA,
    ],
)
def _sc_aggregate(h_hbm, src2d, dst2d, zeros_hbm, part_out,
                  sidx0, sidx1, didx0, didx1,
                  rows0, rows1, rows2, rows3, rows4, acc_sp,
                  gt0, gt1, gt2, gt3, gt4, st0, st1, st2, st3, st4, ix0, ix1):
    c = lax.axis_index("c")
    s = lax.axis_index("s")
    wid = c * NUM_SUBCORES + s
    sidx = (sidx0, sidx1)
    didx = (didx0, didx1)
    rows = (rows0, rows1, rows2, rows3, rows4)
    gts = (gt0, gt1, gt2, gt3, gt4)
    sts = (st0, st1, st2, st3, st4)
    ixs = (ix0, ix1)

    def idx_start(g, slot):
        pltpu.async_copy(src2d.at[wid, pl.ds(g * G, G)], sidx[slot], ixs[slot])
        pltpu.async_copy(dst2d.at[wid, pl.ds(g * G, G)], didx[slot], ixs[slot])

    def idx_wait(g, slot):
        pltpu.make_async_copy(src2d.at[wid, pl.ds(g * G, G)], sidx[slot],
                              ixs[slot]).wait()
        pltpu.make_async_copy(dst2d.at[wid, pl.ds(g * G, G)], didx[slot],
                              ixs[slot]).wait()

    def gth_start(idx_row, p):
        pltpu.async_copy(h_hbm.at[idx_row], rows[p], gts[p])

    def gth_wait(idx_row, p):
        pltpu.make_async_copy(h_hbm.at[idx_row], rows[p], gts[p]).wait()

    def sct_start(idx_row, p):
        pltpu.async_copy(rows[p], acc_sp.at[idx_row], sts[p], add=True)

    def sct_wait(idx_row, p):
        pltpu.make_async_copy(rows[p], acc_sp.at[idx_row], sts[p]).wait()

    pltpu.sync_copy(src2d.at[wid, pl.ds(0, G)], sidx0)
    pltpu.sync_copy(dst2d.at[wid, pl.ds(0, G)], didx0)
    gth_start(sidx0.at[0], 0)
    gth_start(sidx0.at[1], 1)
    gth_start(sidx0.at[2], 2)
    gth_start(sidx0.at[3], 3)
    row0 = s * ROWS_PER_TILE
    pltpu.sync_copy(zeros_hbm.at[pl.ds(row0, ROWS_PER_TILE)],
                    acc_sp.at[pl.ds(row0, ROWS_PER_TILE)])
    plsc.subcore_barrier()

    def tenpack(t, carry):
        for gg in range(10):
            g = 10 * t + gg
            slot, oslot = gg % 2, 1 - gg % 2
            for b in range(G):
                j = g * G + b
                p, p4 = (3 * gg + b) % 5, (3 * gg + b + 4) % 5
                pre = j + 4 < NB
                if b == 0:
                    @pl.when(pre & (j >= 1))
                    def _():
                        sct_wait(didx[oslot].at[G - 1], p4)

                    @pl.when(g + 1 < NG)
                    def _():
                        idx_start(g + 1, oslot)
                else:
                    @pl.when(pre)
                    def _():
                        sct_wait(didx[slot].at[b - 1], p4)

                @pl.when(pre)
                def _():
                    if b >= G - 4:
                        if b == G - 4:
                            idx_wait(g + 1, oslot)
                        gth_start(sidx[oslot].at[b - (G - 4)], p4)
                    else:
                        gth_start(sidx[slot].at[b + 4], p4)

                gth_wait(sidx[slot].at[b], p)
                sct_start(didx[slot].at[b], p)
        return carry

    lax.fori_loop(0, NG // 10, tenpack, 0)
    for i in range(5):
        b = G - 5 + i
        sct_wait(didx[1].at[b], (3 * 9 + b) % 5)
    plsc.subcore_barrier()
    pltpu.sync_copy(acc_sp.at[pl.ds(row0, ROWS_PER_TILE)],
                    part_out.at[c, pl.ds(row0, ROWS_PER_TILE)])


def _dis_from_deg(deg_ref):
    deg = deg_ref[0] + deg_ref[1] + 1.0
    return lax.rsqrt(deg)


def _tc_mm1_body(x_ref, w1_ref, h1_ref):
    h1_ref[...] = jnp.dot(x_ref[...], w1_ref[...],
                          preferred_element_type=jnp.float32)


def _tc_scale_body(deg_ref, h1_ref, h1p_ref):
    deg = deg_ref[0, pl.ds(0, N)] + deg_ref[1, pl.ds(0, N)] + 1.0
    h1p_ref[pl.ds(0, N)] = h1_ref[...] * lax.rsqrt(deg)
    h1p_ref[pl.ds(N, N_PAD - N)] = jnp.zeros((N_PAD - N, D), jnp.float32)


def _tc_mid_body(deg_ref, p_ref, h1p_ref, b1_ref, w2_ref, h2p_ref):
    dis = _dis_from_deg(deg_ref)
    t = (p_ref[0] + p_ref[1] + h1p_ref[...]) * dis + b1_ref[...]
    r = jnp.maximum(t, 0.0)
    h2 = jnp.dot(r, w2_ref[...], preferred_element_type=jnp.float32)
    h2p_ref[...] = h2 * dis


def _tc_post_body(deg_ref, p_ref, h2p_ref, b2_ref, out_ref):
    deg = deg_ref[0, pl.ds(0, N)] + deg_ref[1, pl.ds(0, N)] + 1.0
    dis = lax.rsqrt(deg)
    out_ref[...] = (p_ref[0, pl.ds(0, N)] + p_ref[1, pl.ds(0, N)]
                    + h2p_ref[pl.ds(0, N)]) * dis + b2_ref[...]


_f32 = jnp.float32


def kernel(x, edge_index, W1, b1, W2, b2):
    pad = E_PAD - E
    pad_idx = N + (jnp.arange(pad, dtype=jnp.int32) % PAD_ROWS)
    src2d = jnp.concatenate([edge_index[0], pad_idx]).reshape(NW, NB, BLK)
    dst2d = jnp.concatenate([edge_index[1], pad_idx]).reshape(NW, NB, BLK)
    zeros_big = jnp.zeros((N_PAD, D), _f32)
    iota_nr = jnp.arange(NR, dtype=jnp.int32).reshape(1, NR)
    b1r = b1.reshape(1, D)
    b2r = b2.reshape(1, D)

    deg_parts = _sc_degree(dst2d, zeros_big, iota_nr)
    deg_parts = deg_parts.reshape(NUM_CORES, N_PAD, 1)

    h1 = pl.pallas_call(
        _tc_mm1_body,
        out_shape=jax.ShapeDtypeStruct((N, D), _f32),
    )(x, W1)

    h1p = pl.pallas_call(
        _tc_scale_body,
        out_shape=jax.ShapeDtypeStruct((N_PAD, D), _f32),
    )(deg_parts, h1)

    p1 = _sc_aggregate(h1p, src2d, dst2d, zeros_big)

    h2p = pl.pallas_call(
        _tc_mid_body,
        out_shape=jax.ShapeDtypeStruct((N_PAD, D), _f32),
    )(deg_parts, p1, h1p, b1r, W2)

    p2 = _sc_aggregate(h2p, src2d, dst2d, zeros_big)

    out = pl.pallas_call(
        _tc_post_body,
        out_shape=jax.ShapeDtypeStruct((N, D), _f32),
    )(deg_parts, p2, h2p, b2r)

    return out


# speedup vs baseline: 1.0018x; 1.0018x over previous
import functools

import jax
import jax.numpy as jnp
from jax import lax
from jax.experimental import pallas as pl
from jax.experimental.pallas import tpu as pltpu
from jax.experimental.pallas import tpu_sc as plsc

N = 10000
E = 320000
D = 128

NUM_CORES = 2
NUM_SUBCORES = 16
NW = NUM_CORES * NUM_SUBCORES
BLK = 80
NB = 128
G = 8
NG = NB // G
E_PAD = NW * NB * BLK
PAD_ROWS = 128
N_PAD = 10240
ROWS_PER_TILE = N_PAD // NUM_SUBCORES

_mesh = plsc.VectorSubcoreMesh(core_axis_name="c", subcore_axis_name="s")


NR = N_PAD // 128


@functools.partial(
    pl.kernel,
    out_type=jax.ShapeDtypeStruct((NUM_CORES, NR, 128), jnp.float32),
    mesh=_mesh,
    compiler_params=pltpu.CompilerParams(needs_layout_passes=False),
    scratch_types=[
        pltpu.VMEM((NB, BLK), jnp.int32),
        pltpu.VMEM((NR, 128), jnp.float32),
        pltpu.VMEM((1, NR), jnp.int32),
        pltpu.VMEM_SHARED((NR, 128), jnp.float32),
    ],
)
def _sc_degree(dst2d, zeros_hbm, iota_hbm, deg_out, dst_v, hist_v, iota_v,
               deg_sp):
    c = lax.axis_index("c")
    s = lax.axis_index("s")
    wid = c * NUM_SUBCORES + s
    pltpu.sync_copy(dst2d.at[wid], dst_v)
    pltpu.sync_copy(zeros_hbm.at[pl.ds(0, NR)], hist_v)
    pltpu.sync_copy(iota_hbm, iota_v)

    @pl.when(s == 0)
    def _():
        pltpu.sync_copy(zeros_hbm.at[pl.ds(0, NR)], deg_sp)

    def row(r, carry):
        for k in range(BLK // 16):
            idx = dst_v[r, pl.ds(k * 16, 16)]
            cnt, last = plsc.scan_count(idx)
            plsc.addupdate_scatter(
                hist_v,
                [lax.shift_right_logical(idx, 7), lax.bitwise_and(idx, 127)],
                cnt.astype(jnp.float32), mask=last)
        return carry

    lax.fori_loop(0, NB, row, 0)
    plsc.subcore_barrier()
    pltpu.sync_copy(hist_v, deg_sp.at[iota_v.at[0]], add=True)
    plsc.subcore_barrier()

    @pl.when(s == 0)
    def _():
        pltpu.sync_copy(deg_sp, deg_out.at[c])


@functools.partial(
    pl.kernel,
    out_type=jax.ShapeDtypeStruct((NUM_CORES, N_PAD, D), jnp.float32),
    mesh=_mesh,
    scratch_types=[
        pltpu.VMEM((G, BLK), jnp.int32),
        pltpu.VMEM((G, BLK), jnp.int32),
        pltpu.VMEM((G, BLK), jnp.int32),
        pltpu.VMEM((G, BLK), jnp.int32),
        pltpu.VMEM((BLK, D), jnp.float32),
        pltpu.VMEM((BLK, D), jnp.float32),
        pltpu.VMEM((BLK, D), jnp.float32),
        pltpu.VMEM((BLK, D), jnp.float32),
        pltpu.VMEM_SHARED((N_PAD, D), jnp.float32),
        pltpu.SemaphoreType.DMA,
        pltpu.SemaphoreType.DMA,
        pltpu.SemaphoreType.DMA,
        pltpu.SemaphoreType.DMA,
        pltpu.SemaphoreType.DMA,
        pltpu.SemaphoreType.DMA,
        pltpu.SemaphoreType.DMA,
        pltpu.SemaphoreType.DMA,
        pltpu.SemaphoreType.DMA,
        pltpu.SemaphoreType.DMA,
    ],
)
def _sc_aggregate(h_hbm, src2d, dst2d, zeros_hbm, part_out,
                  sidx0, sidx1, didx0, didx1, rows0, rows1, rows2, rows3,
                  acc_sp, gt0, gt1, gt2, gt3, st0, st1, st2, st3, ix0, ix1):
    c = lax.axis_index("c")
    s = lax.axis_index("s")
    wid = c * NUM_SUBCORES + s
    sidx = (sidx0, sidx1)
    didx = (didx0, didx1)
    rows = (rows0, rows1, rows2, rows3)
    gts = (gt0, gt1, gt2, gt3)
    sts = (st0, st1, st2, st3)
    ixs = (ix0, ix1)

    def idx_start(g, slot):
        pltpu.async_copy(src2d.at[wid, pl.ds(g * G, G)], sidx[slot], ixs[slot])
        pltpu.async_copy(dst2d.at[wid, pl.ds(g * G, G)], didx[slot], ixs[slot])

    def idx_wait(g, slot):
        pltpu.make_async_copy(src2d.at[wid, pl.ds(g * G, G)], sidx[slot],
                              ixs[slot]).wait()
        pltpu.make_async_copy(dst2d.at[wid, pl.ds(g * G, G)], didx[slot],
                              ixs[slot]).wait()

    def gth_start(idx_row, p):
        pltpu.async_copy(h_hbm.at[idx_row], rows[p], gts[p])

    def gth_wait(idx_row, p):
        pltpu.make_async_copy(h_hbm.at[idx_row], rows[p], gts[p]).wait()

    def sct_start(idx_row, p):
        pltpu.async_copy(rows[p], acc_sp.at[idx_row], sts[p], add=True)

    def sct_wait(idx_row, p):
        pltpu.make_async_copy(rows[p], acc_sp.at[idx_row], sts[p]).wait()

    pltpu.sync_copy(src2d.at[wid, pl.ds(0, G)], sidx0)
    pltpu.sync_copy(dst2d.at[wid, pl.ds(0, G)], didx0)
    gth_start(sidx0.at[0], 0)
    gth_start(sidx0.at[1], 1)
    gth_start(sidx0.at[2], 2)
    row0 = s * ROWS_PER_TILE
    pltpu.sync_copy(zeros_hbm.at[pl.ds(row0, ROWS_PER_TILE)],
                    acc_sp.at[pl.ds(row0, ROWS_PER_TILE)])
    plsc.subcore_barrier()

    def pair(t, carry):
        for gg in (0, 1):
            g = 2 * t + gg
            slot, oslot = gg, 1 - gg
            for b in range(G):
                j = g * G + b
                p, p3 = b % 4, (b + 3) % 4
                pre = j + 3 < NB
                if b == 0:
                    @pl.when(pre & (j >= 1))
                    def _():
                        sct_wait(didx[oslot].at[G - 1], p3)

                    @pl.when(g + 1 < NG)
                    def _():
                        idx_start(g + 1, oslot)
                else:
                    @pl.when(pre)
                    def _():
                        sct_wait(didx[slot].at[b - 1], p3)

                @pl.when(pre)
                def _():
                    if b == G - 3:
                        idx_wait(g + 1, oslot)
                        gth_start(sidx[oslot].at[0], p3)
                    elif b == G - 2:
                        gth_start(sidx[oslot].at[1], p3)
                    elif b == G - 1:
                        gth_start(sidx[oslot].at[2], p3)
                    else:
                        gth_start(sidx[slot].at[b + 3], p3)

                gth_wait(sidx[slot].at[b], p)
                sct_start(didx[slot].at[b], p)
        return carry

    lax.fori_loop(0, NG // 2, pair, 0)
    sct_wait(didx[1].at[G - 4], 0)
    sct_wait(didx[1].at[G - 3], 1)
    sct_wait(didx[1].at[G - 2], 2)
    sct_wait(didx[1].at[G - 1], 3)
    plsc.subcore_barrier()
    pltpu.sync_copy(acc_sp.at[pl.ds(row0, ROWS_PER_TILE)],
                    part_out.at[c, pl.ds(row0, ROWS_PER_TILE)])


def _dis_from_deg(deg_ref):
    deg = deg_ref[0] + deg_ref[1] + 1.0
    return lax.rsqrt(deg)


def _tc_mm1_body(x_ref, w1_ref, h1_ref):
    h1_ref[...] = jnp.dot(x_ref[...], w1_ref[...],
                          preferred_element_type=jnp.float32)


def _tc_scale_body(deg_ref, h1_ref, h1p_ref):
    deg = deg_ref[0, pl.ds(0, N)] + deg_ref[1, pl.ds(0, N)] + 1.0
    h1p_ref[pl.ds(0, N)] = h1_ref[...] * lax.rsqrt(deg)
    h1p_ref[pl.ds(N, N_PAD - N)] = jnp.zeros((N_PAD - N, D), jnp.float32)


def _tc_mid_body(deg_ref, p_ref, h1p_ref, b1_ref, w2_ref, h2p_ref):
    dis = _dis_from_deg(deg_ref)
    t = (p_ref[0] + p_ref[1] + h1p_ref[...]) * dis + b1_ref[...]
    r = jnp.maximum(t, 0.0)
    h2 = jnp.dot(r, w2_ref[...], preferred_element_type=jnp.float32)
    h2p_ref[...] = h2 * dis


def _tc_post_body(deg_ref, p_ref, h2p_ref, b2_ref, out_ref):
    deg = deg_ref[0, pl.ds(0, N)] + deg_ref[1, pl.ds(0, N)] + 1.0
    dis = lax.rsqrt(deg)
    out_ref[...] = (p_ref[0, pl.ds(0, N)] + p_ref[1, pl.ds(0, N)]
                    + h2p_ref[pl.ds(0, N)]) * dis + b2_ref[...]


_f32 = jnp.float32


def kernel(x, edge_index, W1, b1, W2, b2):
    pad = E_PAD - E
    pad_idx = N + (jnp.arange(pad, dtype=jnp.int32) % PAD_ROWS)
    src2d = jnp.concatenate([edge_index[0], pad_idx]).reshape(NW, NB, BLK)
    dst2d = jnp.concatenate([edge_index[1], pad_idx]).reshape(NW, NB, BLK)
    zeros_big = jnp.zeros((N_PAD, D), _f32)
    iota_nr = jnp.arange(NR, dtype=jnp.int32).reshape(1, NR)
    b1r = b1.reshape(1, D)
    b2r = b2.reshape(1, D)

    deg_parts = _sc_degree(dst2d, zeros_big, iota_nr)
    deg_parts = deg_parts.reshape(NUM_CORES, N_PAD, 1)

    h1 = pl.pallas_call(
        _tc_mm1_body,
        out_shape=jax.ShapeDtypeStruct((N, D), _f32),
    )(x, W1)

    h1p = pl.pallas_call(
        _tc_scale_body,
        out_shape=jax.ShapeDtypeStruct((N_PAD, D), _f32),
    )(deg_parts, h1)

    p1 = _sc_aggregate(h1p, src2d, dst2d, zeros_big)

    h2p = pl.pallas_call(
        _tc_mid_body,
        out_shape=jax.ShapeDtypeStruct((N_PAD, D), _f32),
    )(deg_parts, p1, h1p, b1r, W2)

    p2 = _sc_aggregate(h2p, src2d, dst2d, zeros_big)

    out = pl.pallas_call(
        _tc_post_body,
        out_shape=jax.ShapeDtypeStruct((N, D), _f32),
    )(deg_parts, p2, h2p, b2r)

    return out
